# baseline (speedup 1.0000x reference)
import functools

import jax
import jax.numpy as jnp
from jax import lax
from jax.experimental import pallas as pl
from jax.experimental.pallas import tpu as pltpu

EPS_ = 1e-10
TILE_R = 128
G1 = 5


def _k1_kernel(row_ref, col_ref, rm_ref, b_ref, num_ref, den_ref, num_acc, den_acc,
               *, h, w, ng):
    g = pl.program_id(0)

    @pl.when(g == 0)
    def _init():
        num_acc[0] = 0.0
        den_acc[0] = 0.0

    x = rm_ref[0]
    bmap = b_ref[0, 0]
    row = row_ref[0, 0]
    col = col_ref[0, 0]

    iota_w = lax.broadcasted_iota(jnp.int32, (TILE_R, 1, w), 2)
    mask_w = iota_w == col[:, None, None]
    iota_h = lax.broadcasted_iota(jnp.int32, (TILE_R, h), 1)
    mask_h = iota_h == row[:, None]

    sum_w = jnp.sum(x, axis=2)
    denom = jnp.sum(sum_w, axis=1)

    srm_w = jnp.sum(jnp.where(mask_w, x, 0.0), axis=2)
    srm = jnp.sum(jnp.where(mask_h, srm_w, 0.0), axis=1)

    sb_w = jnp.sum(jnp.where(mask_w, bmap[None], 0.0), axis=2)
    sb = jnp.sum(jnp.where(mask_h, sb_w, 0.0), axis=1)

    num_acc[0] += jnp.sum(sb * -jnp.log(EPS_ + srm / denom))
    den_acc[0] += jnp.sum(sb)

    @pl.when(g == ng - 1)
    def _fin():
        num_ref[...] = jnp.full((1, 1), num_acc[0], jnp.float32)
        den_ref[...] = jnp.full((1, 1), den_acc[0], jnp.float32)


def _k2_kernel(loc_ref, row_ref, col_ref, xf_ref, b_ref, num_in, den_in,
               out_ref, num_acc, den_acc, *, h, w, hw, ng):
    g = pl.program_id(0)

    @pl.when(g == 0)
    def _init():
        num_acc[0] = num_in[0, 0]
        den_acc[0] = den_in[0, 0]

    x = xf_ref[...]
    bmap = b_ref[0, 0]
    loc = loc_ref[0, 0]
    row = row_ref[0, 0]
    col = col_ref[0, 0]

    iota_f = lax.broadcasted_iota(jnp.int32, (TILE_R, hw), 1)
    mask_f = iota_f == loc[:, None]

    denom = jnp.sum(x, axis=1)
    srm = jnp.sum(jnp.where(mask_f, x, 0.0), axis=1)

    iota_w = lax.broadcasted_iota(jnp.int32, (TILE_R, 1, w), 2)
    mask_w = iota_w == col[:, None, None]
    iota_h = lax.broadcasted_iota(jnp.int32, (TILE_R, h), 1)
    mask_h = iota_h == row[:, None]
    sb_w = jnp.sum(jnp.where(mask_w, bmap[None], 0.0), axis=2)
    sb = jnp.sum(jnp.where(mask_h, sb_w, 0.0), axis=1)

    num_acc[0] += jnp.sum(sb * -jnp.log(EPS_ + srm / denom))
    den_acc[0] += jnp.sum(sb)

    @pl.when(g == ng - 1)
    def _fin():
        out_ref[...] = jnp.full((1, 1), num_acc[0] / (1.0 + den_acc[0]), jnp.float32)


def kernel(response_map, source_feature_1d_locations, boundaries):
    B, S, H, W = response_map.shape
    HW = H * W
    TPB = S // TILE_R
    NG = B * TPB
    G2 = NG - G1

    loc3 = source_feature_1d_locations.astype(jnp.int32).reshape(NG, 1, TILE_R)
    row3 = loc3 // W
    col3 = loc3 % W

    rm_flat = response_map.reshape(B * S, HW)

    num1, den1 = pl.pallas_call(
        functools.partial(_k1_kernel, h=H, w=W, ng=G1),
        grid=(G1,),
        in_specs=[
            pl.BlockSpec((1, 1, TILE_R), lambda g: (g, 0, 0)),
            pl.BlockSpec((1, 1, TILE_R), lambda g: (g, 0, 0)),
            pl.BlockSpec((1, TILE_R, H, W), lambda g: (g // TPB, g % TPB, 0, 0)),
            pl.BlockSpec((1, 1, H, W), lambda g: (g // TPB, 0, 0, 0)),
        ],
        out_specs=[
            pl.BlockSpec((1, 1), lambda g: (0, 0)),
            pl.BlockSpec((1, 1), lambda g: (0, 0)),
        ],
        out_shape=[
            jax.ShapeDtypeStruct((1, 1), jnp.float32),
            jax.ShapeDtypeStruct((1, 1), jnp.float32),
        ],
        scratch_shapes=[
            pltpu.SMEM((1,), jnp.float32),
            pltpu.SMEM((1,), jnp.float32),
        ],
    )(row3, col3, response_map, boundaries)

    out = pl.pallas_call(
        functools.partial(_k2_kernel, h=H, w=W, hw=HW, ng=G2),
        grid=(G2,),
        in_specs=[
            pl.BlockSpec((1, 1, TILE_R), lambda g: (G1 + g, 0, 0)),
            pl.BlockSpec((1, 1, TILE_R), lambda g: (G1 + g, 0, 0)),
            pl.BlockSpec((1, 1, TILE_R), lambda g: (G1 + g, 0, 0)),
            pl.BlockSpec((TILE_R, HW), lambda g: (G1 + g, 0)),
            pl.BlockSpec((1, 1, H, W), lambda g: ((G1 + g) // TPB, 0, 0, 0)),
            pl.BlockSpec((1, 1), lambda g: (0, 0)),
            pl.BlockSpec((1, 1), lambda g: (0, 0)),
        ],
        out_specs=pl.BlockSpec((1, 1), lambda g: (0, 0)),
        out_shape=jax.ShapeDtypeStruct((1, 1), jnp.float32),
        scratch_shapes=[
            pltpu.SMEM((1,), jnp.float32),
            pltpu.SMEM((1,), jnp.float32),
        ],
    )(loc3, row3, col3, rm_flat, boundaries, num1, den1)
    return out[0, 0]


# speedup vs baseline: 1.8700x; 1.8700x over previous
import functools

import jax
import jax.numpy as jnp
from jax import lax
from jax.experimental import pallas as pl
from jax.experimental.pallas import tpu as pltpu

EPS_ = 1e-10
TILE_R = 128


def _loss_kernel(row_ref, col_ref, rm_ref, b_ref, out_ref, num_acc, den_acc,
                 *, h, w, nb, nt):
    b = pl.program_id(0)
    t = pl.program_id(1)

    @pl.when(jnp.logical_and(b == 0, t == 0))
    def _init():
        num_acc[0] = 0.0
        den_acc[0] = 0.0

    x = rm_ref[0]
    bmap = b_ref[0, 0]
    row = row_ref[0, 0]
    col = col_ref[0, 0]

    iota_w = lax.broadcasted_iota(jnp.int32, (TILE_R, 1, w), 2)
    mask_w = iota_w == col[:, None, None]
    iota_h = lax.broadcasted_iota(jnp.int32, (TILE_R, h), 1)
    mask_h = iota_h == row[:, None]

    sum_w = jnp.sum(x, axis=2)
    denom = jnp.sum(sum_w, axis=1)

    srm_w = jnp.sum(jnp.where(mask_w, x, 0.0), axis=2)
    srm = jnp.sum(jnp.where(mask_h, srm_w, 0.0), axis=1)

    maskh_f = jnp.where(mask_h, 1.0, 0.0)
    maskw_f = jnp.where(mask_w[:, 0, :], 1.0, 0.0)
    u = jnp.dot(maskh_f, bmap, preferred_element_type=jnp.float32)
    sb = jnp.sum(u * maskw_f, axis=1)

    num_acc[0] += jnp.sum(sb * -jnp.log(EPS_ + srm / denom))
    den_acc[0] += jnp.sum(sb)

    @pl.when(jnp.logical_and(b == nb - 1, t == nt - 1))
    def _fin():
        out_ref[...] = jnp.full((1, 1), num_acc[0] / (1.0 + den_acc[0]), jnp.float32)


def kernel(response_map, source_feature_1d_locations, boundaries):
    B, S, H, W = response_map.shape
    T = S // TILE_R

    loc = source_feature_1d_locations.astype(jnp.int32)
    row = (loc // W).reshape(B * T, 1, TILE_R)
    col = (loc % W).reshape(B * T, 1, TILE_R)

    out = pl.pallas_call(
        functools.partial(_loss_kernel, h=H, w=W, nb=B, nt=T),
        grid=(B, T),
        in_specs=[
            pl.BlockSpec((1, 1, TILE_R), lambda b, t: (b * T + t, 0, 0)),
            pl.BlockSpec((1, 1, TILE_R), lambda b, t: (b * T + t, 0, 0)),
            pl.BlockSpec((1, TILE_R, H, W), lambda b, t: (b, t, 0, 0)),
            pl.BlockSpec((1, 1, H, W), lambda b, t: (b, 0, 0, 0)),
        ],
        out_specs=pl.BlockSpec((1, 1), lambda b, t: (0, 0)),
        out_shape=jax.ShapeDtypeStruct((1, 1), jnp.float32),
        scratch_shapes=[
            pltpu.SMEM((1,), jnp.float32),
            pltpu.SMEM((1,), jnp.float32),
        ],
    )(row, col, response_map, boundaries)
    return out[0, 0]
